# Initial kernel scaffold; baseline (speedup 1.0000x reference)
import jax
import jax.numpy as jnp
from jax.experimental import pallas as pl


def kernel(x, edge_index, edge_attr, batch, We0, be0, W1_0, b1_0, W2_0, b2_0, We1, be1, W1_1, b1_1, W2_1, b2_1, We2, be2, W1_2, b1_2, W2_2, b2_2, gamma, beta, W3, b3, W4, b4):
    raise NotImplementedError("write your pallas kernel here")


# speedup vs baseline: 1.0187x; 1.0187x over previous
import functools

import jax
import jax.numpy as jnp
from jax.experimental import pallas as pl
from jax.experimental.pallas import tpu as pltpu

N_NODES = 10000
N_EDGES = 320000
NUM_GRAPHS = 256


def _head_body(g_ref, gamma_ref, beta_ref, w3_ref, b3_ref, w4_ref, b4_ref, out_ref):
    g = g_ref[...]
    mean = jnp.mean(g, axis=0, keepdims=True)
    var = jnp.mean((g - mean) ** 2, axis=0, keepdims=True)
    gn = (g - mean) / jnp.sqrt(var + 1e-5) * gamma_ref[...] + beta_ref[...]
    gn = jax.nn.relu(gn)
    h = jax.nn.relu(jnp.dot(gn, w3_ref[...], preferred_element_type=jnp.float32) + b3_ref[...])
    out_ref[...] = jnp.dot(h, w4_ref[...], preferred_element_type=jnp.float32) + b4_ref[...]


def _head(g, gamma, beta, W3, b3, W4, b4):
    return pl.pallas_call(
        _head_body,
        out_shape=jax.ShapeDtypeStruct((NUM_GRAPHS, 12), jnp.float32),
    )(g, gamma.reshape(1, -1), beta.reshape(1, -1), W3, b3.reshape(1, -1), W4, b4.reshape(1, -1))


def kernel(x, edge_index, edge_attr, batch,
           We0, be0, W1_0, b1_0, W2_0, b2_0,
           We1, be1, W1_1, b1_1, W2_1, b2_1,
           We2, be2, W1_2, b1_2, W2_2, b2_2,
           gamma, beta, W3, b3, W4, b4):
    src = edge_index[0]
    dst = edge_index[1]
    h = x
    params = [(We0, be0, W1_0, b1_0, W2_0, b2_0),
              (We1, be1, W1_1, b1_1, W2_1, b2_1),
              (We2, be2, W1_2, b1_2, W2_2, b2_2)]
    for We, be, W1, b1, W2, b2 in params:
        ea = edge_attr @ We + be
        msg = jax.nn.relu(h[src] + ea)
        aggr = jax.ops.segment_sum(msg, dst, num_segments=N_NODES)
        hh = h + aggr
        hh = jax.nn.relu(hh @ W1 + b1)
        hh = hh @ W2 + b2
        h = jax.nn.relu(hh)
    g = jax.ops.segment_sum(h, batch, num_segments=NUM_GRAPHS)
    return _head(g, gamma, beta, W3, b3, W4, b4)


# speedup vs baseline: 1.3952x; 1.3696x over previous
import functools

import jax
import jax.numpy as jnp
from jax import lax
from jax.experimental import pallas as pl
from jax.experimental.pallas import tpu as pltpu
from jax.experimental.pallas import tpu_sc as plsc

N_NODES = 10000
N_EDGES = 320000
NUM_GRAPHS = 256

NC = 2
NS = 16
NW = NC * NS
EPW = N_EDGES // NW
C = 80

_mesh = plsc.VectorSubcoreMesh(core_axis_name="c", subcore_axis_name="s")


def _msg_pass(D):
    @functools.partial(
        pl.kernel,
        out_type=jax.ShapeDtypeStruct((N_EDGES, D), jnp.float32),
        mesh=_mesh,
        scratch_types=[
            pltpu.VMEM((C,), jnp.int32),
            pltpu.VMEM((C, 128), jnp.float32),
            pltpu.VMEM((C, D), jnp.float32),
            pltpu.SemaphoreType.DMA,
            pltpu.SemaphoreType.DMA,
        ],
    )
    def body(h_hbm, ea_hbm, src_hbm, out_hbm, src_v, gbuf, ebuf, sem_g, sem_e):
        c = lax.axis_index("c")
        s = lax.axis_index("s")
        w = s * NC + c
        base_w = w * EPW

        def chunk(i, carry):
            off = base_w + i * C
            pltpu.sync_copy(src_hbm.at[pl.ds(off, C)], src_v)
            cp = pltpu.async_copy(ea_hbm.at[pl.ds(off, C)], ebuf, sem_e)
            pltpu.async_copy(h_hbm.at[src_v], gbuf, sem_g).wait()
            cp.wait()

            def row(e, carry2):
                for j in range(D // 16):
                    v = gbuf[e, pl.ds(j * 16, 16)] + ebuf[e, pl.ds(j * 16, 16)]
                    ebuf[e, pl.ds(j * 16, 16)] = jnp.maximum(v, 0.0)
                return carry2
            lax.fori_loop(0, C, row, 0)
            pltpu.sync_copy(ebuf, out_hbm.at[pl.ds(off, C)])
            return carry

        lax.fori_loop(0, EPW // C, chunk, 0)

    return body


_msg_pass_128 = _msg_pass(128)
_msg_pass_64 = _msg_pass(64)


def _ea_body(attr_ref, w0, b0, w1, b1, w2, b2, o0, o1, o2):
    a = attr_ref[...]
    o0[...] = jnp.dot(a, w0[...], preferred_element_type=jnp.float32) + b0[...]
    o1[...] = jnp.dot(a, w1[...], preferred_element_type=jnp.float32) + b1[...]
    o2[...] = jnp.dot(a, w2[...], preferred_element_type=jnp.float32) + b2[...]


def _ea_all(edge_attr, We0, be0, We1, be1, We2, be2):
    BLK = 2000
    grid = (N_EDGES // BLK,)
    wspec = lambda d: pl.BlockSpec((16, d), lambda i: (0, 0))
    bspec = lambda d: pl.BlockSpec((1, d), lambda i: (0, 0))
    ospec = lambda d: pl.BlockSpec((BLK, d), lambda i: (i, 0))
    return pl.pallas_call(
        _ea_body,
        grid=grid,
        in_specs=[pl.BlockSpec((BLK, 16), lambda i: (i, 0)),
                  wspec(128), bspec(128), wspec(64), bspec(64), wspec(128), bspec(128)],
        out_specs=[ospec(128), ospec(64), ospec(128)],
        out_shape=[jax.ShapeDtypeStruct((N_EDGES, 128), jnp.float32),
                   jax.ShapeDtypeStruct((N_EDGES, 64), jnp.float32),
                   jax.ShapeDtypeStruct((N_EDGES, 128), jnp.float32)],
    )(edge_attr, We0, be0.reshape(1, -1), We1, be1.reshape(1, -1),
      We2, be2.reshape(1, -1))


def _node_body(h_ref, a_ref, w1_ref, b1_ref, w2_ref, b2_ref, o_ref):
    hh = h_ref[...] + a_ref[...]
    t = jax.nn.relu(jnp.dot(hh, w1_ref[...], preferred_element_type=jnp.float32)
                    + b1_ref[...])
    r = jax.nn.relu(jnp.dot(t, w2_ref[...], preferred_element_type=jnp.float32)
                    + b2_ref[...])
    pad = o_ref.shape[1] - r.shape[1]
    if pad:
        r = jnp.concatenate([r, jnp.zeros((r.shape[0], pad), jnp.float32)], axis=1)
    o_ref[...] = r


def _node_mlp(h, aggr, W1, b1, W2, b2, dout_pad=None):
    BLK = 2000
    din, dout = W1.shape
    dp = dout if dout_pad is None else dout_pad
    return pl.pallas_call(
        _node_body,
        grid=(N_NODES // BLK,),
        in_specs=[pl.BlockSpec((BLK, din), lambda i: (i, 0)),
                  pl.BlockSpec((BLK, din), lambda i: (i, 0)),
                  pl.BlockSpec((din, dout), lambda i: (0, 0)),
                  pl.BlockSpec((1, dout), lambda i: (0, 0)),
                  pl.BlockSpec((dout, dout), lambda i: (0, 0)),
                  pl.BlockSpec((1, dout), lambda i: (0, 0))],
        out_specs=pl.BlockSpec((BLK, dp), lambda i: (i, 0)),
        out_shape=jax.ShapeDtypeStruct((N_NODES, dp), jnp.float32),
    )(h, aggr, W1, b1.reshape(1, -1), W2, b2.reshape(1, -1))


def _pool_body(batch_ref, h_ref, g_ref):
    i = pl.program_id(0)

    @pl.when(i == 0)
    def _():
        g_ref[...] = jnp.zeros_like(g_ref)

    b = batch_ref[0, 0, :]
    onehot = (b[:, None] == lax.broadcasted_iota(jnp.int32, (b.shape[0], NUM_GRAPHS), 1)
              ).astype(jnp.float32)
    g_ref[...] += lax.dot_general(onehot, h_ref[...], (((0,), (0,)), ((), ())),
                                  preferred_element_type=jnp.float32,
                                  precision=lax.Precision.HIGHEST)


def _pool(h, batch):
    BLK = 2000
    nb = N_NODES // BLK
    batch3 = batch.reshape(nb, 1, BLK)
    return pl.pallas_call(
        _pool_body,
        grid=(nb,),
        in_specs=[pl.BlockSpec((1, 1, BLK), lambda i: (i, 0, 0)),
                  pl.BlockSpec((BLK, 256), lambda i: (i, 0))],
        out_specs=pl.BlockSpec((NUM_GRAPHS, 256), lambda i: (0, 0)),
        out_shape=jax.ShapeDtypeStruct((NUM_GRAPHS, 256), jnp.float32),
    )(batch3, h)


def _head_body(g_ref, gamma_ref, beta_ref, w3_ref, b3_ref, w4_ref, b4_ref, out_ref):
    g = g_ref[...]
    mean = jnp.mean(g, axis=0, keepdims=True)
    var = jnp.mean((g - mean) ** 2, axis=0, keepdims=True)
    gn = (g - mean) / jnp.sqrt(var + 1e-5) * gamma_ref[...] + beta_ref[...]
    gn = jax.nn.relu(gn)
    t = jax.nn.relu(jnp.dot(gn, w3_ref[...], preferred_element_type=jnp.float32)
                    + b3_ref[...])
    out_ref[...] = jnp.dot(t, w4_ref[...], preferred_element_type=jnp.float32) + b4_ref[...]


def _head(g, gamma, beta, W3, b3, W4, b4):
    return pl.pallas_call(
        _head_body,
        out_shape=jax.ShapeDtypeStruct((NUM_GRAPHS, 12), jnp.float32),
    )(g, gamma.reshape(1, -1), beta.reshape(1, -1), W3, b3.reshape(1, -1),
      W4, b4.reshape(1, -1))


def kernel(x, edge_index, edge_attr, batch,
           We0, be0, W1_0, b1_0, W2_0, b2_0,
           We1, be1, W1_1, b1_1, W2_1, b2_1,
           We2, be2, W1_2, b1_2, W2_2, b2_2,
           gamma, beta, W3, b3, W4, b4):
    src = edge_index[0]
    dst = edge_index[1]
    ea0, ea1, ea2 = _ea_all(edge_attr, We0, be0, We1, be1, We2, be2)

    W1_1p = jnp.concatenate([W1_1, jnp.zeros((64, 128), jnp.float32)], axis=0)

    h = x
    for ea, mp, D, W1, b1, W2, b2, dp in (
            (ea0, _msg_pass_128, 128, W1_0, b1_0, W2_0, b2_0, 128),
            (ea1, _msg_pass_64, 64, W1_1p, b1_1, W2_1, b2_1, None),
            (ea2, _msg_pass_128, 128, W1_2, b1_2, W2_2, b2_2, None)):
        msg = mp(h, ea, src)
        aggr = jax.ops.segment_sum(msg, dst, num_segments=N_NODES)
        if D < 128:
            aggr = jnp.concatenate(
                [aggr, jnp.zeros((N_NODES, 128 - D), jnp.float32)], axis=1)
        h = _node_mlp(h, aggr, W1, b1, W2, b2, dout_pad=dp)

    g = _pool(h, batch)
    return _head(g, gamma, beta, W3, b3, W4, b4)


# speedup vs baseline: 1.5553x; 1.1147x over previous
import functools

import jax
import jax.numpy as jnp
from jax import lax
from jax.experimental import pallas as pl
from jax.experimental.pallas import tpu as pltpu
from jax.experimental.pallas import tpu_sc as plsc

N_NODES = 10000
N_EDGES = 320000
NUM_GRAPHS = 256

NC = 2
NS = 16
NW = NC * NS
EPW = N_EDGES // NW
C = 80

_mesh = plsc.VectorSubcoreMesh(core_axis_name="c", subcore_axis_name="s")


def _msg_pass(D):
    nch = EPW // C

    @functools.partial(
        pl.kernel,
        out_type=jax.ShapeDtypeStruct((N_EDGES, D), jnp.float32),
        mesh=_mesh,
        scratch_types=[
            [pltpu.VMEM((C,), jnp.int32)] * 2,
            [pltpu.VMEM((C, 128), jnp.float32)] * 2,
            [pltpu.VMEM((C, D), jnp.float32)] * 2,
            [pltpu.SemaphoreType.DMA] * 2,
            [pltpu.SemaphoreType.DMA] * 2,
        ],
    )
    def body(h_hbm, ea_hbm, src_hbm, out_hbm, src_v, gbuf, ebuf, sem_g, sem_e):
        c = lax.axis_index("c")
        s = lax.axis_index("s")
        w = s * NC + c
        base_w = w * EPW

        def start(i, b):
            off = base_w + i * C
            pltpu.sync_copy(src_hbm.at[pl.ds(off, C)], src_v[b])
            pltpu.async_copy(ea_hbm.at[pl.ds(off, C)], ebuf[b], sem_e[b])
            pltpu.async_copy(h_hbm.at[src_v[b]], gbuf[b], sem_g[b])

        def finish(i, b):
            off = base_w + i * C
            pltpu.make_async_copy(h_hbm.at[src_v[b]], gbuf[b], sem_g[b]).wait()
            pltpu.make_async_copy(ea_hbm.at[pl.ds(off, C)], ebuf[b], sem_e[b]).wait()

            def row(e, carry2):
                for j in range(D // 16):
                    v = gbuf[b][e, pl.ds(j * 16, 16)] + ebuf[b][e, pl.ds(j * 16, 16)]
                    ebuf[b][e, pl.ds(j * 16, 16)] = jnp.maximum(v, 0.0)
                return carry2
            lax.fori_loop(0, C, row, 0)
            pltpu.sync_copy(ebuf[b], out_hbm.at[pl.ds(off, C)])

        start(0, 0)

        def pair(p, carry):
            for b in range(2):
                i = 2 * p + b
                start(i + 1, 1 - b)
                finish(i, b)
            return carry

        lax.fori_loop(0, (nch - 1) // 2, pair, 0)
        finish(nch - 1, (nch - 1) % 2)

    return body


_msg_pass_128 = _msg_pass(128)
_msg_pass_64 = _msg_pass(64)


def _ea_body(attr_ref, w0, b0, w1, b1, w2, b2, o0, o1, o2):
    a = attr_ref[...]
    o0[...] = jnp.dot(a, w0[...], preferred_element_type=jnp.float32) + b0[...]
    o1[...] = jnp.dot(a, w1[...], preferred_element_type=jnp.float32) + b1[...]
    o2[...] = jnp.dot(a, w2[...], preferred_element_type=jnp.float32) + b2[...]


def _ea_all(edge_attr, We0, be0, We1, be1, We2, be2):
    BLK = 2000
    grid = (N_EDGES // BLK,)
    wspec = lambda d: pl.BlockSpec((16, d), lambda i: (0, 0))
    bspec = lambda d: pl.BlockSpec((1, d), lambda i: (0, 0))
    ospec = lambda d: pl.BlockSpec((BLK, d), lambda i: (i, 0))
    return pl.pallas_call(
        _ea_body,
        grid=grid,
        in_specs=[pl.BlockSpec((BLK, 16), lambda i: (i, 0)),
                  wspec(128), bspec(128), wspec(64), bspec(64), wspec(128), bspec(128)],
        out_specs=[ospec(128), ospec(64), ospec(128)],
        out_shape=[jax.ShapeDtypeStruct((N_EDGES, 128), jnp.float32),
                   jax.ShapeDtypeStruct((N_EDGES, 64), jnp.float32),
                   jax.ShapeDtypeStruct((N_EDGES, 128), jnp.float32)],
    )(edge_attr, We0, be0.reshape(1, -1), We1, be1.reshape(1, -1),
      We2, be2.reshape(1, -1))


def _node_body(h_ref, a_ref, w1_ref, b1_ref, w2_ref, b2_ref, o_ref):
    hh = h_ref[...] + a_ref[...]
    t = jax.nn.relu(jnp.dot(hh, w1_ref[...], preferred_element_type=jnp.float32)
                    + b1_ref[...])
    r = jax.nn.relu(jnp.dot(t, w2_ref[...], preferred_element_type=jnp.float32)
                    + b2_ref[...])
    pad = o_ref.shape[1] - r.shape[1]
    if pad:
        r = jnp.concatenate([r, jnp.zeros((r.shape[0], pad), jnp.float32)], axis=1)
    o_ref[...] = r


def _node_mlp(h, aggr, W1, b1, W2, b2, dout_pad=None):
    BLK = 2000
    din, dout = W1.shape
    dp = dout if dout_pad is None else dout_pad
    return pl.pallas_call(
        _node_body,
        grid=(N_NODES // BLK,),
        in_specs=[pl.BlockSpec((BLK, din), lambda i: (i, 0)),
                  pl.BlockSpec((BLK, din), lambda i: (i, 0)),
                  pl.BlockSpec((din, dout), lambda i: (0, 0)),
                  pl.BlockSpec((1, dout), lambda i: (0, 0)),
                  pl.BlockSpec((dout, dout), lambda i: (0, 0)),
                  pl.BlockSpec((1, dout), lambda i: (0, 0))],
        out_specs=pl.BlockSpec((BLK, dp), lambda i: (i, 0)),
        out_shape=jax.ShapeDtypeStruct((N_NODES, dp), jnp.float32),
    )(h, aggr, W1, b1.reshape(1, -1), W2, b2.reshape(1, -1))


def _pool_body(batch_ref, h_ref, g_ref):
    i = pl.program_id(0)

    @pl.when(i == 0)
    def _():
        g_ref[...] = jnp.zeros_like(g_ref)

    b = batch_ref[0, 0, :]
    onehot = (b[:, None] == lax.broadcasted_iota(jnp.int32, (b.shape[0], NUM_GRAPHS), 1)
              ).astype(jnp.float32)
    g_ref[...] += lax.dot_general(onehot, h_ref[...], (((0,), (0,)), ((), ())),
                                  preferred_element_type=jnp.float32,
                                  precision=lax.Precision.HIGHEST)


def _pool(h, batch):
    BLK = 2000
    nb = N_NODES // BLK
    batch3 = batch.reshape(nb, 1, BLK)
    return pl.pallas_call(
        _pool_body,
        grid=(nb,),
        in_specs=[pl.BlockSpec((1, 1, BLK), lambda i: (i, 0, 0)),
                  pl.BlockSpec((BLK, 256), lambda i: (i, 0))],
        out_specs=pl.BlockSpec((NUM_GRAPHS, 256), lambda i: (0, 0)),
        out_shape=jax.ShapeDtypeStruct((NUM_GRAPHS, 256), jnp.float32),
    )(batch3, h)


def _head_body(g_ref, gamma_ref, beta_ref, w3_ref, b3_ref, w4_ref, b4_ref, out_ref):
    g = g_ref[...]
    mean = jnp.mean(g, axis=0, keepdims=True)
    var = jnp.mean((g - mean) ** 2, axis=0, keepdims=True)
    gn = (g - mean) / jnp.sqrt(var + 1e-5) * gamma_ref[...] + beta_ref[...]
    gn = jax.nn.relu(gn)
    t = jax.nn.relu(jnp.dot(gn, w3_ref[...], preferred_element_type=jnp.float32)
                    + b3_ref[...])
    out_ref[...] = jnp.dot(t, w4_ref[...], preferred_element_type=jnp.float32) + b4_ref[...]


def _head(g, gamma, beta, W3, b3, W4, b4):
    return pl.pallas_call(
        _head_body,
        out_shape=jax.ShapeDtypeStruct((NUM_GRAPHS, 12), jnp.float32),
    )(g, gamma.reshape(1, -1), beta.reshape(1, -1), W3, b3.reshape(1, -1),
      W4, b4.reshape(1, -1))


def kernel(x, edge_index, edge_attr, batch,
           We0, be0, W1_0, b1_0, W2_0, b2_0,
           We1, be1, W1_1, b1_1, W2_1, b2_1,
           We2, be2, W1_2, b1_2, W2_2, b2_2,
           gamma, beta, W3, b3, W4, b4):
    src = edge_index[0]
    dst = edge_index[1]
    ea0, ea1, ea2 = _ea_all(edge_attr, We0, be0, We1, be1, We2, be2)

    W1_1p = jnp.concatenate([W1_1, jnp.zeros((64, 128), jnp.float32)], axis=0)

    h = x
    for ea, mp, D, W1, b1, W2, b2, dp in (
            (ea0, _msg_pass_128, 128, W1_0, b1_0, W2_0, b2_0, 128),
            (ea1, _msg_pass_64, 64, W1_1p, b1_1, W2_1, b2_1, None),
            (ea2, _msg_pass_128, 128, W1_2, b1_2, W2_2, b2_2, None)):
        msg = mp(h, ea, src)
        aggr = jax.ops.segment_sum(msg, dst, num_segments=N_NODES)
        if D < 128:
            aggr = jnp.concatenate(
                [aggr, jnp.zeros((N_NODES, 128 - D), jnp.float32)], axis=1)
        h = _node_mlp(h, aggr, W1, b1, W2, b2, dout_pad=dp)

    g = _pool(h, batch)
    return _head(g, gamma, beta, W3, b3, W4, b4)


# speedup vs baseline: 1.5602x; 1.0032x over previous
import functools

import jax
import jax.numpy as jnp
from jax import lax
from jax.experimental import pallas as pl
from jax.experimental.pallas import tpu as pltpu
from jax.experimental.pallas import tpu_sc as plsc

N_NODES = 10000
N_EDGES = 320000
NUM_GRAPHS = 256

NC = 2
NS = 16
NW = NC * NS
EPW = N_EDGES // NW
C = 80

_mesh = plsc.VectorSubcoreMesh(core_axis_name="c", subcore_axis_name="s")


def _msg_pass(D):
    nch = EPW // C

    @functools.partial(
        pl.kernel,
        out_type=jax.ShapeDtypeStruct((N_EDGES, D), jnp.float32),
        mesh=_mesh,
        scratch_types=[
            [pltpu.VMEM((C,), jnp.int32)] * 4,
            [pltpu.VMEM((C, 128), jnp.float32)] * 4,
            [pltpu.VMEM((C, D), jnp.float32)] * 4,
            [pltpu.SemaphoreType.DMA] * 4,
            [pltpu.SemaphoreType.DMA] * 4,
        ],
    )
    def body(h_hbm, ea_hbm, src_hbm, out_hbm, src_v, gbuf, ebuf, sem_g, sem_e):
        c = lax.axis_index("c")
        s = lax.axis_index("s")
        w = s * NC + c
        base_w = w * EPW

        def start(i, b):
            off = base_w + i * C
            pltpu.sync_copy(src_hbm.at[pl.ds(off, C)], src_v[b])
            pltpu.async_copy(ea_hbm.at[pl.ds(off, C)], ebuf[b], sem_e[b])
            pltpu.async_copy(h_hbm.at[src_v[b]], gbuf[b], sem_g[b])

        def finish(i, b):
            off = base_w + i * C
            pltpu.make_async_copy(h_hbm.at[src_v[b]], gbuf[b], sem_g[b]).wait()
            pltpu.make_async_copy(ea_hbm.at[pl.ds(off, C)], ebuf[b], sem_e[b]).wait()

            def row(e, carry2):
                for j in range(D // 16):
                    v = gbuf[b][e, pl.ds(j * 16, 16)] + ebuf[b][e, pl.ds(j * 16, 16)]
                    ebuf[b][e, pl.ds(j * 16, 16)] = jnp.maximum(v, 0.0)
                return carry2
            lax.fori_loop(0, C, row, 0)
            pltpu.sync_copy(ebuf[b], out_hbm.at[pl.ds(off, C)])

        start(0, 0)
        start(1, 1)
        start(2, 2)

        def quad(p, carry):
            for b in range(4):
                i = 4 * p + b
                start(i + 3, (b + 3) % 4)
                finish(i, b)
            return carry

        lax.fori_loop(0, (nch - 5) // 4, quad, 0)
        start(nch - 2, (nch - 2) % 4)
        finish(nch - 5, (nch - 5) % 4)
        start(nch - 1, (nch - 1) % 4)
        finish(nch - 4, (nch - 4) % 4)
        finish(nch - 3, (nch - 3) % 4)
        finish(nch - 2, (nch - 2) % 4)
        finish(nch - 1, (nch - 1) % 4)

    return body


_msg_pass_128 = _msg_pass(128)
_msg_pass_64 = _msg_pass(64)


def _ea_body(attr_ref, w0, b0, w1, b1, w2, b2, o0, o1, o2):
    a = attr_ref[...]
    o0[...] = jnp.dot(a, w0[...], preferred_element_type=jnp.float32) + b0[...]
    o1[...] = jnp.dot(a, w1[...], preferred_element_type=jnp.float32) + b1[...]
    o2[...] = jnp.dot(a, w2[...], preferred_element_type=jnp.float32) + b2[...]


def _ea_all(edge_attr, We0, be0, We1, be1, We2, be2):
    BLK = 2000
    grid = (N_EDGES // BLK,)
    wspec = lambda d: pl.BlockSpec((16, d), lambda i: (0, 0))
    bspec = lambda d: pl.BlockSpec((1, d), lambda i: (0, 0))
    ospec = lambda d: pl.BlockSpec((BLK, d), lambda i: (i, 0))
    return pl.pallas_call(
        _ea_body,
        grid=grid,
        in_specs=[pl.BlockSpec((BLK, 16), lambda i: (i, 0)),
                  wspec(128), bspec(128), wspec(64), bspec(64), wspec(128), bspec(128)],
        out_specs=[ospec(128), ospec(64), ospec(128)],
        out_shape=[jax.ShapeDtypeStruct((N_EDGES, 128), jnp.float32),
                   jax.ShapeDtypeStruct((N_EDGES, 64), jnp.float32),
                   jax.ShapeDtypeStruct((N_EDGES, 128), jnp.float32)],
    )(edge_attr, We0, be0.reshape(1, -1), We1, be1.reshape(1, -1),
      We2, be2.reshape(1, -1))


def _node_body(h_ref, a_ref, w1_ref, b1_ref, w2_ref, b2_ref, o_ref):
    hh = h_ref[...] + a_ref[...]
    t = jax.nn.relu(jnp.dot(hh, w1_ref[...], preferred_element_type=jnp.float32)
                    + b1_ref[...])
    r = jax.nn.relu(jnp.dot(t, w2_ref[...], preferred_element_type=jnp.float32)
                    + b2_ref[...])
    pad = o_ref.shape[1] - r.shape[1]
    if pad:
        r = jnp.concatenate([r, jnp.zeros((r.shape[0], pad), jnp.float32)], axis=1)
    o_ref[...] = r


def _node_mlp(h, aggr, W1, b1, W2, b2, dout_pad=None):
    BLK = 2000
    din, dout = W1.shape
    dp = dout if dout_pad is None else dout_pad
    return pl.pallas_call(
        _node_body,
        grid=(N_NODES // BLK,),
        in_specs=[pl.BlockSpec((BLK, din), lambda i: (i, 0)),
                  pl.BlockSpec((BLK, din), lambda i: (i, 0)),
                  pl.BlockSpec((din, dout), lambda i: (0, 0)),
                  pl.BlockSpec((1, dout), lambda i: (0, 0)),
                  pl.BlockSpec((dout, dout), lambda i: (0, 0)),
                  pl.BlockSpec((1, dout), lambda i: (0, 0))],
        out_specs=pl.BlockSpec((BLK, dp), lambda i: (i, 0)),
        out_shape=jax.ShapeDtypeStruct((N_NODES, dp), jnp.float32),
    )(h, aggr, W1, b1.reshape(1, -1), W2, b2.reshape(1, -1))


def _pool_body(batch_ref, h_ref, g_ref):
    i = pl.program_id(0)

    @pl.when(i == 0)
    def _():
        g_ref[...] = jnp.zeros_like(g_ref)

    b = batch_ref[0, 0, :]
    onehot = (b[:, None] == lax.broadcasted_iota(jnp.int32, (b.shape[0], NUM_GRAPHS), 1)
              ).astype(jnp.float32)
    g_ref[...] += lax.dot_general(onehot, h_ref[...], (((0,), (0,)), ((), ())),
                                  preferred_element_type=jnp.float32,
                                  precision=lax.Precision.HIGHEST)


def _pool(h, batch):
    BLK = 2000
    nb = N_NODES // BLK
    batch3 = batch.reshape(nb, 1, BLK)
    return pl.pallas_call(
        _pool_body,
        grid=(nb,),
        in_specs=[pl.BlockSpec((1, 1, BLK), lambda i: (i, 0, 0)),
                  pl.BlockSpec((BLK, 256), lambda i: (i, 0))],
        out_specs=pl.BlockSpec((NUM_GRAPHS, 256), lambda i: (0, 0)),
        out_shape=jax.ShapeDtypeStruct((NUM_GRAPHS, 256), jnp.float32),
    )(batch3, h)


def _head_body(g_ref, gamma_ref, beta_ref, w3_ref, b3_ref, w4_ref, b4_ref, out_ref):
    g = g_ref[...]
    mean = jnp.mean(g, axis=0, keepdims=True)
    var = jnp.mean((g - mean) ** 2, axis=0, keepdims=True)
    gn = (g - mean) / jnp.sqrt(var + 1e-5) * gamma_ref[...] + beta_ref[...]
    gn = jax.nn.relu(gn)
    t = jax.nn.relu(jnp.dot(gn, w3_ref[...], preferred_element_type=jnp.float32)
                    + b3_ref[...])
    out_ref[...] = jnp.dot(t, w4_ref[...], preferred_element_type=jnp.float32) + b4_ref[...]


def _head(g, gamma, beta, W3, b3, W4, b4):
    return pl.pallas_call(
        _head_body,
        out_shape=jax.ShapeDtypeStruct((NUM_GRAPHS, 12), jnp.float32),
    )(g, gamma.reshape(1, -1), beta.reshape(1, -1), W3, b3.reshape(1, -1),
      W4, b4.reshape(1, -1))


def kernel(x, edge_index, edge_attr, batch,
           We0, be0, W1_0, b1_0, W2_0, b2_0,
           We1, be1, W1_1, b1_1, W2_1, b2_1,
           We2, be2, W1_2, b1_2, W2_2, b2_2,
           gamma, beta, W3, b3, W4, b4):
    src = edge_index[0]
    dst = edge_index[1]
    ea0, ea1, ea2 = _ea_all(edge_attr, We0, be0, We1, be1, We2, be2)

    W1_1p = jnp.concatenate([W1_1, jnp.zeros((64, 128), jnp.float32)], axis=0)

    h = x
    for ea, mp, D, W1, b1, W2, b2, dp in (
            (ea0, _msg_pass_128, 128, W1_0, b1_0, W2_0, b2_0, 128),
            (ea1, _msg_pass_64, 64, W1_1p, b1_1, W2_1, b2_1, None),
            (ea2, _msg_pass_128, 128, W1_2, b1_2, W2_2, b2_2, None)):
        msg = mp(h, ea, src)
        aggr = jax.ops.segment_sum(msg, dst, num_segments=N_NODES)
        if D < 128:
            aggr = jnp.concatenate(
                [aggr, jnp.zeros((N_NODES, 128 - D), jnp.float32)], axis=1)
        h = _node_mlp(h, aggr, W1, b1, W2, b2, dout_pad=dp)

    g = _pool(h, batch)
    return _head(g, gamma, beta, W3, b3, W4, b4)
